# baseline (speedup 1.0000x reference)
import functools
import numpy as np
import jax
import jax.numpy as jnp
from jax import lax
from jax.experimental import pallas as pl
from jax.experimental.pallas import tpu as pltpu
from jax.experimental.pallas import tpu_sc as plsc

_THETA = 0.3
_ALPHA = 0.2
_BETA = 0.7
_RHO1 = 0.9
_NCLS = 1000
_K = 100

_ROWS = 16384
_BLK = 512
_GRID = _ROWS // _BLK

_NW = 32
_RPW = _ROWS // _NW
_RC = 16
_NCHUNK = _RPW // _RC
_NB = 512
_BLO = -8.0
_BWID = 16.0 / _NB
_NSL = _NCLS // 16


def _sc_thresh_body(logits_hbm, out_hbm, row_v, hist_v, thr_v):
    c = lax.axis_index("c")
    s = lax.axis_index("s")
    wid = s * 2 + c
    lane = lax.iota(jnp.int32, 16)
    ones = jnp.ones((16,), jnp.float32)
    scale = jnp.float32(1.0 / _BWID)
    blo = jnp.float32(_BLO)

    def bucketize(v):
        b = ((v - blo) * scale).astype(jnp.int32)
        return jnp.clip(b, 0, _NB - 1)

    def chunk_body(ch, _):
        row0 = wid * _RPW + ch * _RC
        pltpu.sync_copy(logits_hbm.at[pl.ds(row0, _RC), :], row_v)

        def row_body(r, thrv):
            def clr(j, carry):
                hist_v[pl.ds(j * 16, 16)] = jnp.zeros((16,), jnp.float32)
                return carry

            lax.fori_loop(0, _NB // 16, clr, 0)

            def slice_body(i, carry):
                v = row_v[r, pl.ds(i * 16, 16)]
                plsc.addupdate_scatter(hist_v, [bucketize(v)], ones)
                return carry

            lax.fori_loop(0, _NSL, slice_body, 0)
            vt = row_v[r, pl.ds(_NCLS - 16, 16)]
            plsc.addupdate_scatter(hist_v, [bucketize(vt)], ones,
                                   mask=lane >= 8)

            def scan_body(j, carry):
                run, bk = carry
                hv = hist_v[pl.ds(j * 16, 16)]
                cum = plsc.cumsum(hv) + run
                m = cum >= jnp.float32(_K)
                npos = plsc.all_reduce_population_count(m)
                ffs = plsc.all_reduce_ffs(m)
                hit = (bk < 0) & (npos > 0)
                bk = jnp.where(hit, j * 16 + ffs, bk)
                run = run + jnp.sum(hv)
                return (run, bk)

            _, bk = lax.fori_loop(
                0, _NB // 16, scan_body,
                (jnp.float32(0.0), jnp.full((16,), -1, jnp.int32)))
            thr = blo + (bk + 1).astype(jnp.float32) * jnp.float32(_BWID)
            return jnp.where(lane == r, thr, thrv)

        thrv = lax.fori_loop(0, _RC, row_body, jnp.zeros((16,), jnp.float32))
        thr_v[...] = thrv
        pltpu.sync_copy(thr_v, out_hbm.at[pl.ds(row0, _RC)])
        return 0

    lax.fori_loop(0, _NCHUNK, chunk_body, 0)


def _sc_thresholds(logits):
    mesh = plsc.VectorSubcoreMesh(core_axis_name="c", subcore_axis_name="s")
    kfn = functools.partial(
        pl.kernel,
        mesh=mesh,
        out_type=jax.ShapeDtypeStruct((_ROWS,), jnp.float32),
        scratch_types=[
            pltpu.VMEM((_RC, _NCLS), jnp.float32),
            pltpu.VMEM((_NB,), jnp.float32),
            pltpu.VMEM((16,), jnp.float32),
        ],
        compiler_params=pltpu.CompilerParams(needs_layout_passes=False),
    )(_sc_thresh_body)
    return kfn(logits)


def _main_body(logits_ref, targets_ref, thr_ref, out_ref):
    l = logits_ref[...]
    t = targets_ref[...]
    sel = l <= thr_ref[...]

    p = jax.nn.sigmoid(l)
    nlp = -jnp.log(p + 1e-7)
    nl1p = -jnp.log((1.0 - p) + 1e-7)
    ent = p * nlp + (1.0 - p) * nl1p
    pos_term = _BETA * ((1.0 - _RHO1) * nl1p + _RHO1 * nlp)
    unk_term = -_ALPHA * ent
    branch = jnp.where(sel, 0.0, jnp.where(p > _THETA, pos_term, unk_term))
    out_ref[0, 0, 0] = jnp.sum(t * nlp + (1.0 - t) * branch)


def _warm_body(logits_ref, targets_ref, out_ref):
    l = logits_ref[...]
    t = targets_ref[...]
    p = jax.nn.sigmoid(l)
    nlp = -jnp.log(p + 1e-7)
    nl1p = -jnp.log((1.0 - p) + 1e-7)
    ent = p * nlp + (1.0 - p) * nl1p
    out_ref[0, 0, 0] = jnp.sum(t * nlp - (1.0 - t) * _ALPHA * ent)


_OUT_SPECS = dict(
    out_specs=pl.BlockSpec((1, 1, 1), lambda i: (i, 0, 0),
                           memory_space=pltpu.SMEM),
    out_shape=jax.ShapeDtypeStruct((_GRID, 1, 1), jnp.float32),
    compiler_params=pltpu.CompilerParams(dimension_semantics=("parallel",)),
)


def _run_main(logits, targets):
    thr = _sc_thresholds(logits).reshape(_ROWS, 1)
    partials = pl.pallas_call(
        _main_body,
        grid=(_GRID,),
        in_specs=[
            pl.BlockSpec((_BLK, _NCLS), lambda i: (i, 0)),
            pl.BlockSpec((_BLK, _NCLS), lambda i: (i, 0)),
            pl.BlockSpec((_BLK, 1), lambda i: (i, 0)),
        ],
        **_OUT_SPECS,
    )(logits, targets, thr)
    return jnp.sum(partials)


def _run_warm(logits, targets):
    partials = pl.pallas_call(
        _warm_body,
        grid=(_GRID,),
        in_specs=[
            pl.BlockSpec((_BLK, _NCLS), lambda i: (i, 0)),
            pl.BlockSpec((_BLK, _NCLS), lambda i: (i, 0)),
        ],
        **_OUT_SPECS,
    )(logits, targets)
    return jnp.sum(partials)


def kernel(logits, targets, epoch):
    loss = jax.lax.cond(
        epoch > 0,
        lambda: _run_main(logits, targets),
        lambda: _run_warm(logits, targets),
    )
    return (loss, targets)
